# baseline (speedup 1.0000x reference)
import functools

import jax
import jax.numpy as jnp
from jax.experimental import pallas as pl
from jax.experimental.pallas import tpu as pltpu

_LN_EPS = 1e-5
_INV_SQRT2 = 0.7071067811865476

_ERF_CLAMP = 3.832506856900711
_ERF_ALPHA = (0.00022905065861350646, 0.0034082910107109506,
              0.050955695062380861, 0.18520832239976145, 1.128379143519084)
_ERF_BETA = (-1.1791602954361697e-07, 2.3547966471313185e-05,
             0.0010179625278914885, 0.014070470171167667,
             0.11098505178285362, 0.49746925110067538, 1.0)


def _erf(x):
    x = jnp.clip(x, -_ERF_CLAMP, _ERF_CLAMP)
    t = x * x
    num = jnp.float32(_ERF_ALPHA[0])
    for c in _ERF_ALPHA[1:]:
        num = num * t + jnp.float32(c)
    den = jnp.float32(_ERF_BETA[0])
    for c in _ERF_BETA[1:]:
        den = den * t + jnp.float32(c)
    return x * num / den


def _col_count(mask):
    return jnp.sum(mask.astype(jnp.float32), axis=0, keepdims=True)


def _topk_keep_t(a, k):
    n, bb = a.shape
    kf = jnp.float32(k)
    key = jax.lax.bitcast_convert_type(a, jnp.int32)
    key = jnp.where(key < 0, key ^ jnp.int32(0x7FFFFFFF), key)

    lo0 = jnp.full((1, bb), jnp.iinfo(jnp.int32).min, jnp.int32)
    hi0 = jnp.full((1, bb), jnp.iinfo(jnp.int32).max, jnp.int32)

    def vstep(_, carry):
        lo, hi = carry
        xh = lo ^ hi
        mid = (lo & hi) + (xh >> 1) + (xh & 1)
        cnt = _col_count(key >= mid)
        ge = cnt >= kf
        return jnp.where(ge, mid, lo), jnp.where(ge, hi, mid - 1)

    t, _ = jax.lax.fori_loop(0, 32, vstep, (lo0, hi0))

    gt = key > t
    eq = key == t
    m = kf - _col_count(gt)
    iota = jax.lax.broadcasted_iota(jnp.int32, (n, bb), 0)

    clo0 = jnp.zeros((1, bb), jnp.int32)
    chi0 = jnp.full((1, bb), n, jnp.int32)

    def istep(_, carry):
        clo, chi = carry
        cmid = (clo + chi) >> 1
        cc = _col_count(eq & (iota < cmid))
        geq = cc >= m
        return jnp.where(geq, clo, cmid + 1), jnp.where(geq, cmid, chi)

    _, c = jax.lax.fori_loop(0, 12, istep, (clo0, chi0))
    return gt | (eq & (iota < c))


def _population_t(h, g, b, k):
    mu = jnp.mean(h, axis=0, keepdims=True)
    d = h - mu
    var = jnp.mean(d * d, axis=0, keepdims=True)
    hn = d * jax.lax.rsqrt(var + _LN_EPS) * g + b
    a = hn * (_erf(hn * _INV_SQRT2) + 1.0) * 0.5
    keep = _topk_keep_t(a, k)
    return jnp.where(keep, a, 0.0)


def _mask_cast_body(w_ref, m_ref, o_ref):
    o_ref[...] = (w_ref[...] * m_ref[...]).astype(jnp.bfloat16)


def _column_body(x_ref, w_in_ref, b_in_ref, g_ref, bt_ref, ffm_ref, ffb_ref,
                 w_out_ref, b_out_ref, o_ref, *, k, nlayers):
    c_nt = (((1,), (1,)), ((), ()))
    c_nn = (((1,), (0,)), ((), ()))
    h = jax.lax.dot_general(w_in_ref[...], x_ref[...].astype(jnp.bfloat16),
                            c_nt, preferred_element_type=jnp.float32)
    h = h + b_in_ref[...]
    for l in range(nlayers - 1):
        a = _population_t(h, g_ref[:, l:l + 1], bt_ref[:, l:l + 1], k)
        h = jax.lax.dot_general(ffm_ref[l], a.astype(jnp.bfloat16), c_nn,
                                preferred_element_type=jnp.float32)
        h = h + ffb_ref[:, l:l + 1]
    a = _population_t(h, g_ref[:, nlayers - 1:nlayers],
                      bt_ref[:, nlayers - 1:nlayers], k)
    out = jax.lax.dot_general(w_out_ref[...], a.astype(jnp.bfloat16), c_nn,
                              preferred_element_type=jnp.float32) \
        + b_out_ref[...]
    o_ref[...] = out.T


def kernel(x, W_in, b_in, ln_scale, ln_bias, ff_w, ff_b, ff_mask, W_out,
           b_out):
    B, D = x.shape
    N = W_in.shape[0]
    L = ln_scale.shape[0]
    Lm = ff_w.shape[0]
    k = max(1, int(0.1 * N))
    Bb = 256
    RB = 256

    ffm = pl.pallas_call(
        _mask_cast_body,
        grid=(Lm, N // RB),
        in_specs=[
            pl.BlockSpec((1, RB, N), lambda l, r: (l, r, 0)),
            pl.BlockSpec((1, RB, N), lambda l, r: (l, r, 0)),
        ],
        out_specs=pl.BlockSpec((1, RB, N), lambda l, r: (l, r, 0)),
        out_shape=jax.ShapeDtypeStruct((Lm, N, N), jnp.bfloat16),
        compiler_params=pltpu.CompilerParams(
            dimension_semantics=("parallel", "parallel")),
    )(ff_w, ff_mask)

    return pl.pallas_call(
        functools.partial(_column_body, k=k, nlayers=L),
        grid=(B // Bb,),
        in_specs=[
            pl.BlockSpec((Bb, D), lambda i: (i, 0)),
            pl.BlockSpec((N, D), lambda i: (0, 0)),
            pl.BlockSpec((N, 1), lambda i: (0, 0)),
            pl.BlockSpec((N, L), lambda i: (0, 0)),
            pl.BlockSpec((N, L), lambda i: (0, 0)),
            pl.BlockSpec((Lm, N, N), lambda i: (0, 0, 0)),
            pl.BlockSpec((N, Lm), lambda i: (0, 0)),
            pl.BlockSpec((N, N), lambda i: (0, 0)),
            pl.BlockSpec((N, 1), lambda i: (0, 0)),
        ],
        out_specs=pl.BlockSpec((Bb, N), lambda i: (i, 0)),
        out_shape=jax.ShapeDtypeStruct((B, N), jnp.float32),
        compiler_params=pltpu.CompilerParams(
            dimension_semantics=("parallel",),
            vmem_limit_bytes=100 * 1024 * 1024,
        ),
    )(x, W_in.astype(jnp.bfloat16), b_in.reshape(N, 1), ln_scale.T,
      ln_bias.T, ffm, ff_b.T, W_out.astype(jnp.bfloat16), b_out.reshape(N, 1))


# speedup vs baseline: 1.2167x; 1.2167x over previous
import functools

import jax
import jax.numpy as jnp
from jax.experimental import pallas as pl
from jax.experimental.pallas import tpu as pltpu

_LN_EPS = 1e-5
_INV_SQRT2 = 0.7071067811865476

_ERF_CLAMP = 3.832506856900711
_ERF_ALPHA = (0.00022905065861350646, 0.0034082910107109506,
              0.050955695062380861, 0.18520832239976145, 1.128379143519084)
_ERF_BETA = (-1.1791602954361697e-07, 2.3547966471313185e-05,
             0.0010179625278914885, 0.014070470171167667,
             0.11098505178285362, 0.49746925110067538, 1.0)


def _erf(x):
    x = jnp.clip(x, -_ERF_CLAMP, _ERF_CLAMP)
    t = x * x
    num = jnp.float32(_ERF_ALPHA[0])
    for c in _ERF_ALPHA[1:]:
        num = num * t + jnp.float32(c)
    den = jnp.float32(_ERF_BETA[0])
    for c in _ERF_BETA[1:]:
        den = den * t + jnp.float32(c)
    return x * num / den


def _sum0(ones):
    n = ones.shape[0]
    while n > 16:
        n //= 2
        ones = ones[:n] + ones[n:]
    s = ones.astype(jnp.float32)
    s = s[:8] + s[8:]
    return jnp.sum(s, axis=0, keepdims=True)


def _sort16_as_bf(x32):
    bits = jnp.where(x32 < 0, x32 ^ jnp.int32(0x7FFF), x32)
    return jax.lax.bitcast_convert_type(bits.astype(jnp.int16), jnp.bfloat16)


def _topk_keep_mul_t(a, k, iota16, iota32):
    n, bb = a.shape
    kf = jnp.float32(k)
    bf_one = jnp.ones((), jnp.bfloat16)
    bf_zero = jnp.zeros((), jnp.bfloat16)
    key = jax.lax.bitcast_convert_type(a, jnp.int32)
    key = jnp.where(key < 0, key ^ jnp.int32(0x7FFFFFFF), key)
    ab = _sort16_as_bf(key >> 16)
    lowb = (key ^ jnp.int32(0x8000)).astype(jnp.int16)

    lo0 = jnp.full((1, bb), -32768, jnp.int32)
    hi0 = jnp.full((1, bb), 32767, jnp.int32)

    def astep(_, carry):
        lo, hi = carry
        mid = (lo + hi + 1) >> 1
        midc = jnp.clip(mid, jnp.int32(-32641), jnp.int32(32640))
        cnt = _sum0(jnp.where(ab >= _sort16_as_bf(midc), bf_one, bf_zero))
        ge = cnt >= kf
        return jnp.where(ge, mid, lo), jnp.where(ge, hi, mid - 1)

    t16, _ = jax.lax.fori_loop(0, 16, astep, (lo0, hi0))
    t16b = _sort16_as_bf(t16)
    eqA = ab == t16b
    onesA = jnp.where(eqA, bf_one, bf_zero)
    m = kf - _sum0(jnp.where(ab > t16b, bf_one, bf_zero))

    blo0 = jnp.full((1, bb), -32768, jnp.int32)
    bhi0 = jnp.full((1, bb), 32767, jnp.int32)

    def bstep(_, carry):
        lo, hi = carry
        mid = (lo + hi + 1) >> 1
        cnt = _sum0(jnp.where(lowb >= mid.astype(jnp.int16), onesA, bf_zero))
        ge = cnt >= m
        return jnp.where(ge, mid, lo), jnp.where(ge, hi, mid - 1)

    tB, _ = jax.lax.fori_loop(0, 16, bstep, (blo0, bhi0))
    tb16 = tB.astype(jnp.int16)
    onesAB = jnp.where(lowb == tb16, onesA, bf_zero)
    m3 = m - _sum0(jnp.where(lowb > tb16, onesA, bf_zero))

    clo0 = jnp.zeros((1, bb), jnp.int32)
    chi0 = jnp.full((1, bb), n, jnp.int32)

    def cstep(_, carry):
        clo, chi = carry
        cmid = (clo + chi) >> 1
        cc = _sum0(jnp.where(iota16 < cmid.astype(jnp.int16), onesAB,
                             bf_zero))
        geq = cc >= m3
        return jnp.where(geq, clo, cmid + 1), jnp.where(geq, cmid, chi)

    _, c = jax.lax.fori_loop(0, 12, cstep, (clo0, chi0))

    tlo_u = (tB & jnp.int32(0xFFFF)) ^ jnp.int32(0x8000)
    t32 = (t16 << 16) | tlo_u
    zero = jnp.float32(0.0)
    return jnp.where(
        key > t32, a,
        jnp.where(key == t32, jnp.where(iota32 < c, a, zero), zero))


def _population_t(h, g, b, k, iota16, iota32):
    mu = jnp.mean(h, axis=0, keepdims=True)
    d = h - mu
    var = jnp.mean(d * d, axis=0, keepdims=True)
    hn = d * jax.lax.rsqrt(var + _LN_EPS) * g + b
    a = hn * (_erf(hn * _INV_SQRT2) + 1.0) * 0.5
    return _topk_keep_mul_t(a, k, iota16, iota32)


def _mask_cast_body(w_ref, m_ref, o_ref):
    o_ref[...] = (w_ref[...] * m_ref[...]).astype(jnp.bfloat16)


def _column_body(x_ref, w_in_ref, b_in_ref, g_ref, bt_ref, ffm_ref, ffb_ref,
                 w_out_ref, b_out_ref, o_ref, *, k, nlayers):
    c_nt = (((1,), (1,)), ((), ()))
    c_nn = (((1,), (0,)), ((), ()))
    n, bb = w_in_ref.shape[0], x_ref.shape[0]
    iota32 = jax.lax.broadcasted_iota(jnp.int32, (n, bb), 0)
    iota16 = iota32.astype(jnp.int16)
    h = jax.lax.dot_general(w_in_ref[...], x_ref[...].astype(jnp.bfloat16),
                            c_nt, preferred_element_type=jnp.float32)
    h = h + b_in_ref[...]
    for l in range(nlayers - 1):
        a = _population_t(h, g_ref[:, l:l + 1], bt_ref[:, l:l + 1], k,
                          iota16, iota32)
        h = jax.lax.dot_general(ffm_ref[l], a.astype(jnp.bfloat16), c_nn,
                                preferred_element_type=jnp.float32)
        h = h + ffb_ref[:, l:l + 1]
    a = _population_t(h, g_ref[:, nlayers - 1:nlayers],
                      bt_ref[:, nlayers - 1:nlayers], k, iota16, iota32)
    out = jax.lax.dot_general(w_out_ref[...], a.astype(jnp.bfloat16), c_nn,
                              preferred_element_type=jnp.float32) \
        + b_out_ref[...]
    o_ref[...] = out.T


def kernel(x, W_in, b_in, ln_scale, ln_bias, ff_w, ff_b, ff_mask, W_out,
           b_out):
    B, D = x.shape
    N = W_in.shape[0]
    L = ln_scale.shape[0]
    Lm = ff_w.shape[0]
    k = max(1, int(0.1 * N))
    Bb = 256
    RB = 256

    ffm = pl.pallas_call(
        _mask_cast_body,
        grid=(Lm, N // RB),
        in_specs=[
            pl.BlockSpec((1, RB, N), lambda l, r: (l, r, 0)),
            pl.BlockSpec((1, RB, N), lambda l, r: (l, r, 0)),
        ],
        out_specs=pl.BlockSpec((1, RB, N), lambda l, r: (l, r, 0)),
        out_shape=jax.ShapeDtypeStruct((Lm, N, N), jnp.bfloat16),
        compiler_params=pltpu.CompilerParams(
            dimension_semantics=("parallel", "parallel")),
    )(ff_w, ff_mask)

    return pl.pallas_call(
        functools.partial(_column_body, k=k, nlayers=L),
        grid=(B // Bb,),
        in_specs=[
            pl.BlockSpec((Bb, D), lambda i: (i, 0)),
            pl.BlockSpec((N, D), lambda i: (0, 0)),
            pl.BlockSpec((N, 1), lambda i: (0, 0)),
            pl.BlockSpec((N, L), lambda i: (0, 0)),
            pl.BlockSpec((N, L), lambda i: (0, 0)),
            pl.BlockSpec((Lm, N, N), lambda i: (0, 0, 0)),
            pl.BlockSpec((N, Lm), lambda i: (0, 0)),
            pl.BlockSpec((N, N), lambda i: (0, 0)),
            pl.BlockSpec((N, 1), lambda i: (0, 0)),
        ],
        out_specs=pl.BlockSpec((Bb, N), lambda i: (i, 0)),
        out_shape=jax.ShapeDtypeStruct((B, N), jnp.float32),
        compiler_params=pltpu.CompilerParams(
            dimension_semantics=("parallel",),
            vmem_limit_bytes=100 * 1024 * 1024,
        ),
    )(x, W_in.astype(jnp.bfloat16), b_in.reshape(N, 1), ln_scale.T,
      ln_bias.T, ffm, ff_b.T, W_out.astype(jnp.bfloat16), b_out.reshape(N, 1))
